# baseline (speedup 1.0000x reference)
import jax
import jax.numpy as jnp
from jax import lax
from jax.experimental import pallas as pl
from jax.experimental.pallas import tpu as pltpu
from jax.experimental.pallas import tpu_sc as plsc

NUM_WORKERS = 32
CHUNK = 128
D = 64


def _gather_body(x_hbm, w_hbm, out_hbm, idx_v, rows_v, sem):
  n_chunks = idx_v.shape[0]
  c = lax.axis_index("c")
  s = lax.axis_index("s")
  wid = s * 2 + c
  pltpu.sync_copy(x_hbm.at[wid], idx_v)

  def step(j, carry):
    pltpu.async_copy(w_hbm.at[idx_v.at[j]], rows_v, sem).wait()
    pltpu.sync_copy(rows_v, out_hbm.at[wid, j])
    return carry

  lax.fori_loop(0, n_chunks, step, 0)


def kernel(x, weight):
  B = x.shape[0] * x.shape[1]
  n_chunks = B // (NUM_WORKERS * CHUNK)
  xr = x.reshape(NUM_WORKERS, n_chunks, CHUNK).astype(jnp.int32)

  mesh = plsc.VectorSubcoreMesh(core_axis_name="c", subcore_axis_name="s")
  run = pl.kernel(
      _gather_body,
      out_type=jax.ShapeDtypeStruct(
          (NUM_WORKERS, n_chunks, CHUNK, D), jnp.float32),
      mesh=mesh,
      scratch_types=[
          pltpu.VMEM((n_chunks, CHUNK), jnp.int32),
          pltpu.VMEM((CHUNK, D), jnp.float32),
          pltpu.SemaphoreType.DMA,
      ],
      compiler_params=pltpu.CompilerParams(use_tc_tiling_on_sc=False),
  )
  out = run(xr, weight)
  return out.reshape(x.shape[0], x.shape[1], D)


# speedup vs baseline: 1.0412x; 1.0412x over previous
import jax
import jax.numpy as jnp
from jax import lax
from jax.experimental import pallas as pl
from jax.experimental.pallas import tpu as pltpu
from jax.experimental.pallas import tpu_sc as plsc

NUM_WORKERS = 32
CHUNK = 128
K = 5
D = 64


def _gather_body(x_hbm, w_hbm, out_hbm, idx_v, buf_a, buf_b, gsem_a, gsem_b,
                 wsem_a, wsem_b):
  n_chunks = idx_v.shape[0]
  ng = n_chunks // K
  c = lax.axis_index("c")
  s = lax.axis_index("s")
  wid = s * 2 + c
  pltpu.sync_copy(x_hbm.at[wid], idx_v)

  def start_gathers(g, buf, sem):
    for b in range(K):
      pltpu.async_copy(w_hbm.at[idx_v.at[g * K + b]], buf.at[b], sem)

  def wait_gathers(g, buf, sem):
    for b in range(K):
      pltpu.make_async_copy(w_hbm.at[idx_v.at[g * K + b]], buf.at[b],
                            sem).wait()

  def start_writes(g, buf, sem):
    for b in range(K):
      pltpu.async_copy(buf.at[b], out_hbm.at[wid, g * K + b], sem)

  def wait_writes(g, buf, sem):
    for b in range(K):
      pltpu.make_async_copy(buf.at[b], out_hbm.at[wid, g * K + b],
                            sem).wait()

  start_gathers(0, buf_a, gsem_a)
  wait_gathers(0, buf_a, gsem_a)
  start_writes(0, buf_a, wsem_a)
  start_gathers(1, buf_b, gsem_b)
  wait_gathers(1, buf_b, gsem_b)
  start_writes(1, buf_b, wsem_b)
  wait_writes(0, buf_a, wsem_a)
  start_gathers(2, buf_a, gsem_a)

  def pair(q, carry):
    g = 2 * q + 2
    wait_gathers(g, buf_a, gsem_a)
    start_writes(g, buf_a, wsem_a)
    wait_writes(g - 1, buf_b, wsem_b)
    start_gathers(g + 1, buf_b, gsem_b)
    wait_gathers(g + 1, buf_b, gsem_b)
    start_writes(g + 1, buf_b, wsem_b)
    wait_writes(g, buf_a, wsem_a)
    start_gathers(g + 2, buf_a, gsem_a)
    return carry

  lax.fori_loop(0, (ng - 4) // 2, pair, 0)

  wait_gathers(ng - 2, buf_a, gsem_a)
  start_writes(ng - 2, buf_a, wsem_a)
  wait_writes(ng - 3, buf_b, wsem_b)
  start_gathers(ng - 1, buf_b, gsem_b)
  wait_gathers(ng - 1, buf_b, gsem_b)
  start_writes(ng - 1, buf_b, wsem_b)
  wait_writes(ng - 2, buf_a, wsem_a)
  wait_writes(ng - 1, buf_b, wsem_b)


def kernel(x, weight):
  B = x.shape[0] * x.shape[1]
  n_chunks = B // (NUM_WORKERS * CHUNK)
  xr = x.reshape(NUM_WORKERS, n_chunks, CHUNK).astype(jnp.int32)

  mesh = plsc.VectorSubcoreMesh(core_axis_name="c", subcore_axis_name="s")
  run = pl.kernel(
      _gather_body,
      out_type=jax.ShapeDtypeStruct(
          (NUM_WORKERS, n_chunks, CHUNK, D), jnp.float32),
      mesh=mesh,
      scratch_types=[
          pltpu.VMEM((n_chunks, CHUNK), jnp.int32),
          pltpu.VMEM((K, CHUNK, D), jnp.float32),
          pltpu.VMEM((K, CHUNK, D), jnp.float32),
          pltpu.SemaphoreType.DMA,
          pltpu.SemaphoreType.DMA,
          pltpu.SemaphoreType.DMA,
          pltpu.SemaphoreType.DMA,
      ],
      compiler_params=pltpu.CompilerParams(use_tc_tiling_on_sc=False),
  )
  out = run(xr, weight)
  return out.reshape(x.shape[0], x.shape[1], D)
